# baseline (speedup 1.0000x reference)
import functools

import jax
import jax.numpy as jnp
from jax import lax
from jax.experimental import pallas as pl
from jax.experimental.pallas import tpu as pltpu
from jax.experimental.pallas import tpu_sc as plsc

ROWS = 128
COLS = 65536
HALF = COLS // 2
LANES = 16
SUB = 8
R_SC = 64
R_TC = ROWS - R_SC
NQ = 2
QCOLS = HALF // NQ
CHUNK_C = 1024
NBUF = 4
NCHUNKS = QCOLS // CHUNK_C
NEG_INF = float("-inf")
BIG_I32 = 2**31 - 1

TC_RB = 32
TC_NR = R_TC // TC_RB
TC_NACC = 4
TC_UNROLL = 8


def _shuffle(x, idx):
  dnums = lax.GatherDimensionNumbers(
      offset_dims=(), collapsed_slice_dims=(0,), start_index_map=(0,))
  return lax.gather(
      x, idx[:, None], dimension_numbers=dnums, slice_sizes=(1,),
      mode=lax.GatherScatterMode.PROMISE_IN_BOUNDS)


def _take(mx, bi, omx, obi):
  take = (omx > mx) | ((omx == mx) & (obi < bi))
  return lax.select(take, omx, mx), lax.select(take, obi, bi)


def _lane_argmax(mx, bi, iota):
  for sh in (8, 4, 2, 1):
    idx = iota ^ sh
    mx, bi = _take(mx, bi, _shuffle(mx, idx), _shuffle(bi, idx))
  return bi


def _sc_body(x_hbm, start_hbm, end_hbm,
             b0, b1, b2, b3, mx_buf, bi_buf, res_buf,
             sh_mx, sh_bi, s0, s1, s2, s3):
  cid = lax.axis_index("c")
  sid = lax.axis_index("s")
  rblock = sid // NQ
  q = sid % NQ
  row0 = rblock * SUB
  col0 = cid * HALF + q * QCOLS
  qcol0 = q * QCOLS

  iota = lax.iota(jnp.int32, LANES)
  bufs = (b0, b1, b2, b3)
  sems = (s0, s1, s2, s3)

  def issue(c, b):
    pltpu.async_copy(
        x_hbm.at[pl.ds(row0, SUB), pl.ds(col0 + c * CHUNK_C, CHUNK_C)],
        bufs[b], sems[b])

  def drain(b):
    pltpu.make_async_copy(
        x_hbm.at[pl.ds(row0, SUB), pl.ds(col0, CHUNK_C)],
        bufs[b], sems[b]).wait()

  for b in range(NBUF):
    issue(b, b)

  neg = jnp.full((LANES,), NEG_INF, jnp.float32)
  zero = jnp.zeros((LANES,), jnp.int32)

  def chunk_fold(buf, chunk_col, mxs, bis):
    def step(g, carry):
      mxs_c, bis_c = carry
      new_mx = list(mxs_c)
      new_bi = list(bis_c)
      col = g * LANES
      cur = iota + (chunk_col + col)
      for s in range(SUB):
        v = buf[s, pl.ds(col, LANES)]
        m = v > new_mx[s]
        new_mx[s] = lax.select(m, v, new_mx[s])
        new_bi[s] = lax.select(m, cur, new_bi[s])
      return tuple(new_mx), tuple(new_bi)

    return lax.fori_loop(0, CHUNK_C // LANES, step, (mxs, bis))

  n_rounds = NCHUNKS // NBUF - 1

  def round_body(r, carry):
    mxs, bis = carry
    for b in range(NBUF):
      c = r * NBUF + b
      drain(b)
      mxs, bis = chunk_fold(bufs[b], qcol0 + c * CHUNK_C, mxs, bis)
      issue(c + NBUF, b)
    return mxs, bis

  mxs, bis = lax.fori_loop(
      0, n_rounds, round_body,
      (tuple([neg] * SUB), tuple([zero] * SUB)))

  for b in range(NBUF):
    c = (NCHUNKS - NBUF) + b
    drain(b)
    mxs, bis = chunk_fold(bufs[b], qcol0 + c * CHUNK_C, mxs, bis)
  mxs, bis = list(mxs), list(bis)

  for s in range(SUB):
    mx_buf[pl.ds(s * LANES, LANES)] = mxs[s]
    bi_buf[pl.ds(s * LANES, LANES)] = bis[s]
  slot = SUB * LANES
  pltpu.sync_copy(mx_buf, sh_mx.at[pl.ds(sid * slot, slot)])
  pltpu.sync_copy(bi_buf, sh_bi.at[pl.ds(sid * slot, slot)])
  plsc.subcore_barrier()

  @pl.when(q == 0)
  def _():
    mx_l = list(mxs)
    bi_l = list(bis)
    for dq in range(1, NQ):
      pltpu.sync_copy(sh_mx.at[pl.ds((sid + dq) * slot, slot)], mx_buf)
      pltpu.sync_copy(sh_bi.at[pl.ds((sid + dq) * slot, slot)], bi_buf)
      for s in range(SUB):
        mx_l[s], bi_l[s] = _take(
            mx_l[s], bi_l[s],
            mx_buf[pl.ds(s * LANES, LANES)],
            bi_buf[pl.ds(s * LANES, LANES)])
    res_v = jnp.zeros((LANES,), jnp.int32)
    for s in range(SUB):
      idx_v = _lane_argmax(mx_l[s], bi_l[s], iota)
      res_v = lax.select(iota == s, idx_v, res_v)
    res_buf[...] = res_v

    @pl.when(cid == 0)
    def _():
      pltpu.sync_copy(res_buf.at[pl.ds(0, SUB)],
                      start_hbm.at[pl.ds(row0, SUB)])

    @pl.when(cid == 1)
    def _():
      pltpu.sync_copy(res_buf.at[pl.ds(0, SUB)],
                      end_hbm.at[pl.ds(row0, SUB)])


def _sc_argmax(x):
  mesh = plsc.VectorSubcoreMesh(core_axis_name="c", subcore_axis_name="s")
  run = functools.partial(
      pl.kernel,
      out_type=(jax.ShapeDtypeStruct((R_SC,), jnp.int32),
                jax.ShapeDtypeStruct((R_SC,), jnp.int32)),
      mesh=mesh,
      scratch_types=(
          [pltpu.VMEM((SUB, CHUNK_C), jnp.float32)] * NBUF
          + [pltpu.VMEM((SUB * LANES,), jnp.float32),
             pltpu.VMEM((SUB * LANES,), jnp.int32),
             pltpu.VMEM((LANES,), jnp.int32),
             pltpu.VMEM_SHARED((16 * SUB * LANES,), jnp.float32),
             pltpu.VMEM_SHARED((16 * SUB * LANES,), jnp.int32)]
          + [pltpu.SemaphoreType.DMA] * NBUF
      ),
      compiler_params=pltpu.CompilerParams(use_tc_tiling_on_sc=True),
  )(_sc_body)
  return run(x)


def _tc_kernel(x_ref, out_ref):
  lane = lax.broadcasted_iota(jnp.int32, (TC_RB, 128), 1)
  neg = jnp.full((TC_RB, 128), NEG_INF, jnp.float32)
  zero = jnp.zeros((TC_RB, 128), jnp.int32)

  def step(i, carry):
    mxs, bis = carry
    new_mx = list(mxs)
    new_bi = list(bis)
    for u in range(TC_UNROLL):
      a = u % TC_NACC
      v = i * TC_UNROLL + u
      val = x_ref[:, pl.ds(v * 128, 128)]
      cur = lane + v * 128
      m = val > new_mx[a]
      new_mx[a] = jnp.where(m, val, new_mx[a])
      new_bi[a] = jnp.where(m, cur, new_bi[a])
    return tuple(new_mx), tuple(new_bi)

  n_iters = HALF // 128 // TC_UNROLL
  mxs, bis = lax.fori_loop(
      0, n_iters, step, (tuple([neg] * TC_NACC), tuple([zero] * TC_NACC)))

  mx, bi = mxs[0], bis[0]
  for a in range(1, TC_NACC):
    take = (mxs[a] > mx) | ((mxs[a] == mx) & (bis[a] < bi))
    mx = jnp.where(take, mxs[a], mx)
    bi = jnp.where(take, bis[a], bi)
  mbest = jnp.max(mx, axis=1, keepdims=True)
  cand = jnp.where(mx == mbest, bi, BIG_I32)
  idx = jnp.min(cand, axis=1).astype(jnp.int32)
  out_ref[...] = idx.reshape(1, 1, 1, TC_RB)


def _tc_argmax(x):
  return pl.pallas_call(
      _tc_kernel,
      grid=(2, TC_NR),
      in_specs=[pl.BlockSpec(
          (TC_RB, HALF),
          lambda h, r: (r + R_SC // TC_RB, h))],
      out_specs=pl.BlockSpec((1, 1, 1, TC_RB), lambda h, r: (h, r, 0, 0)),
      out_shape=jax.ShapeDtypeStruct((2, TC_NR, 1, TC_RB), jnp.int32),
  )(x)


@jax.jit
def _argmax_halves(x):
  sc_start, sc_end = _sc_argmax(x)
  tc_out = _tc_argmax(x)
  start = jnp.concatenate([sc_start, tc_out[0].reshape(R_TC)])
  end = jnp.concatenate([sc_end, tc_out[1].reshape(R_TC)])
  return start, end


def kernel(inputs):
  start, end = _argmax_halves(inputs)
  return (start, end)


# speedup vs baseline: 1.0053x; 1.0053x over previous
import functools

import jax
import jax.numpy as jnp
from jax import lax
from jax.experimental import pallas as pl
from jax.experimental.pallas import tpu as pltpu
from jax.experimental.pallas import tpu_sc as plsc

ROWS = 128
COLS = 65536
HALF = COLS // 2
LANES = 16
SUB = 8
R_SC = 32
R_TC = ROWS - R_SC
NQ = 4
QCOLS = HALF // NQ
CHUNK_C = 1024
NBUF = 4
NCHUNKS = QCOLS // CHUNK_C
NEG_INF = float("-inf")
BIG_I32 = 2**31 - 1

TC_RB = 32
TC_NR = R_TC // TC_RB
TC_NACC = 4
TC_UNROLL = 8


def _shuffle(x, idx):
  dnums = lax.GatherDimensionNumbers(
      offset_dims=(), collapsed_slice_dims=(0,), start_index_map=(0,))
  return lax.gather(
      x, idx[:, None], dimension_numbers=dnums, slice_sizes=(1,),
      mode=lax.GatherScatterMode.PROMISE_IN_BOUNDS)


def _take(mx, bi, omx, obi):
  take = (omx > mx) | ((omx == mx) & (obi < bi))
  return lax.select(take, omx, mx), lax.select(take, obi, bi)


def _lane_argmax(mx, bi, iota):
  for sh in (8, 4, 2, 1):
    idx = iota ^ sh
    mx, bi = _take(mx, bi, _shuffle(mx, idx), _shuffle(bi, idx))
  return bi


def _sc_body(x_hbm, start_hbm, end_hbm,
             b0, b1, b2, b3, mx_buf, bi_buf, res_buf,
             sh_mx, sh_bi, s0, s1, s2, s3):
  cid = lax.axis_index("c")
  sid = lax.axis_index("s")
  rblock = sid // NQ
  q = sid % NQ
  row0 = rblock * SUB
  col0 = cid * HALF + q * QCOLS
  qcol0 = q * QCOLS

  iota = lax.iota(jnp.int32, LANES)
  bufs = (b0, b1, b2, b3)
  sems = (s0, s1, s2, s3)

  def issue(c, b):
    pltpu.async_copy(
        x_hbm.at[pl.ds(row0, SUB), pl.ds(col0 + c * CHUNK_C, CHUNK_C)],
        bufs[b], sems[b])

  def drain(b):
    pltpu.make_async_copy(
        x_hbm.at[pl.ds(row0, SUB), pl.ds(col0, CHUNK_C)],
        bufs[b], sems[b]).wait()

  for b in range(NBUF):
    issue(b, b)

  neg = jnp.full((LANES,), NEG_INF, jnp.float32)
  zero = jnp.zeros((LANES,), jnp.int32)

  def chunk_fold(buf, chunk_col, mxs, bis):
    def step(g, carry):
      mxs_c, bis_c = carry
      new_mx = list(mxs_c)
      new_bi = list(bis_c)
      col = g * LANES
      cur = iota + (chunk_col + col)
      for s in range(SUB):
        v = buf[s, pl.ds(col, LANES)]
        m = v > new_mx[s]
        new_mx[s] = lax.select(m, v, new_mx[s])
        new_bi[s] = lax.select(m, cur, new_bi[s])
      return tuple(new_mx), tuple(new_bi)

    return lax.fori_loop(0, CHUNK_C // LANES, step, (mxs, bis))

  n_rounds = NCHUNKS // NBUF - 1

  def round_body(r, carry):
    mxs, bis = carry
    for b in range(NBUF):
      c = r * NBUF + b
      drain(b)
      mxs, bis = chunk_fold(bufs[b], qcol0 + c * CHUNK_C, mxs, bis)
      issue(c + NBUF, b)
    return mxs, bis

  mxs, bis = lax.fori_loop(
      0, n_rounds, round_body,
      (tuple([neg] * SUB), tuple([zero] * SUB)))

  for b in range(NBUF):
    c = (NCHUNKS - NBUF) + b
    drain(b)
    mxs, bis = chunk_fold(bufs[b], qcol0 + c * CHUNK_C, mxs, bis)
  mxs, bis = list(mxs), list(bis)

  for s in range(SUB):
    mx_buf[pl.ds(s * LANES, LANES)] = mxs[s]
    bi_buf[pl.ds(s * LANES, LANES)] = bis[s]
  slot = SUB * LANES
  pltpu.sync_copy(mx_buf, sh_mx.at[pl.ds(sid * slot, slot)])
  pltpu.sync_copy(bi_buf, sh_bi.at[pl.ds(sid * slot, slot)])
  plsc.subcore_barrier()

  @pl.when(q == 0)
  def _():
    mx_l = list(mxs)
    bi_l = list(bis)
    for dq in range(1, NQ):
      pltpu.sync_copy(sh_mx.at[pl.ds((sid + dq) * slot, slot)], mx_buf)
      pltpu.sync_copy(sh_bi.at[pl.ds((sid + dq) * slot, slot)], bi_buf)
      for s in range(SUB):
        mx_l[s], bi_l[s] = _take(
            mx_l[s], bi_l[s],
            mx_buf[pl.ds(s * LANES, LANES)],
            bi_buf[pl.ds(s * LANES, LANES)])
    res_v = jnp.zeros((LANES,), jnp.int32)
    for s in range(SUB):
      idx_v = _lane_argmax(mx_l[s], bi_l[s], iota)
      res_v = lax.select(iota == s, idx_v, res_v)
    res_buf[...] = res_v

    @pl.when(cid == 0)
    def _():
      pltpu.sync_copy(res_buf.at[pl.ds(0, SUB)],
                      start_hbm.at[pl.ds(row0, SUB)])

    @pl.when(cid == 1)
    def _():
      pltpu.sync_copy(res_buf.at[pl.ds(0, SUB)],
                      end_hbm.at[pl.ds(row0, SUB)])


def _sc_argmax(x):
  mesh = plsc.VectorSubcoreMesh(core_axis_name="c", subcore_axis_name="s")
  run = functools.partial(
      pl.kernel,
      out_type=(jax.ShapeDtypeStruct((R_SC,), jnp.int32),
                jax.ShapeDtypeStruct((R_SC,), jnp.int32)),
      mesh=mesh,
      scratch_types=(
          [pltpu.VMEM((SUB, CHUNK_C), jnp.float32)] * NBUF
          + [pltpu.VMEM((SUB * LANES,), jnp.float32),
             pltpu.VMEM((SUB * LANES,), jnp.int32),
             pltpu.VMEM((LANES,), jnp.int32),
             pltpu.VMEM_SHARED((16 * SUB * LANES,), jnp.float32),
             pltpu.VMEM_SHARED((16 * SUB * LANES,), jnp.int32)]
          + [pltpu.SemaphoreType.DMA] * NBUF
      ),
      compiler_params=pltpu.CompilerParams(use_tc_tiling_on_sc=True, skip_device_barrier=True),
  )(_sc_body)
  return run(x)


def _tc_kernel(x_ref, out_ref):
  lane = lax.broadcasted_iota(jnp.int32, (TC_RB, 128), 1)
  neg = jnp.full((TC_RB, 128), NEG_INF, jnp.float32)
  zero = jnp.zeros((TC_RB, 128), jnp.int32)

  def step(i, carry):
    mxs, bis = carry
    new_mx = list(mxs)
    new_bi = list(bis)
    for u in range(TC_UNROLL):
      a = u % TC_NACC
      v = i * TC_UNROLL + u
      val = x_ref[:, pl.ds(v * 128, 128)]
      cur = lane + v * 128
      m = val > new_mx[a]
      new_mx[a] = jnp.where(m, val, new_mx[a])
      new_bi[a] = jnp.where(m, cur, new_bi[a])
    return tuple(new_mx), tuple(new_bi)

  n_iters = HALF // 128 // TC_UNROLL
  mxs, bis = lax.fori_loop(
      0, n_iters, step, (tuple([neg] * TC_NACC), tuple([zero] * TC_NACC)))

  mx, bi = mxs[0], bis[0]
  for a in range(1, TC_NACC):
    take = (mxs[a] > mx) | ((mxs[a] == mx) & (bis[a] < bi))
    mx = jnp.where(take, mxs[a], mx)
    bi = jnp.where(take, bis[a], bi)
  mbest = jnp.max(mx, axis=1, keepdims=True)
  cand = jnp.where(mx == mbest, bi, BIG_I32)
  idx = jnp.min(cand, axis=1).astype(jnp.int32)
  out_ref[...] = idx.reshape(1, 1, 1, TC_RB)


def _tc_argmax(x):
  return pl.pallas_call(
      _tc_kernel,
      grid=(2, TC_NR),
      in_specs=[pl.BlockSpec(
          (TC_RB, HALF),
          lambda h, r: (r + R_SC // TC_RB, h))],
      out_specs=pl.BlockSpec((1, 1, 1, TC_RB), lambda h, r: (h, r, 0, 0)),
      out_shape=jax.ShapeDtypeStruct((2, TC_NR, 1, TC_RB), jnp.int32),
  )(x)


@jax.jit
def _argmax_halves(x):
  sc_start, sc_end = _sc_argmax(x)
  tc_out = _tc_argmax(x)
  start = jnp.concatenate([sc_start, tc_out[0].reshape(R_TC)])
  end = jnp.concatenate([sc_end, tc_out[1].reshape(R_TC)])
  return start, end


def kernel(inputs):
  start, end = _argmax_halves(inputs)
  return (start, end)


# speedup vs baseline: 1.0066x; 1.0013x over previous
import functools

import jax
import jax.numpy as jnp
from jax import lax
from jax.experimental import pallas as pl
from jax.experimental.pallas import tpu as pltpu
from jax.experimental.pallas import tpu_sc as plsc

ROWS = 128
COLS = 65536
HALF = COLS // 2
LANES = 16
SUB = 8
R_SC = 32
R_TC = ROWS - R_SC
NQ = 4
QCOLS = HALF // NQ
CHUNK_C = 1024
NBUF = 4
NCHUNKS = QCOLS // CHUNK_C
NEG_INF = float("-inf")
BIG_I32 = 2**31 - 1

TC_RB = 32
TC_NR = R_TC // TC_RB
TC_NACC = 4
TC_UNROLL = 8


def _shuffle(x, idx):
  dnums = lax.GatherDimensionNumbers(
      offset_dims=(), collapsed_slice_dims=(0,), start_index_map=(0,))
  return lax.gather(
      x, idx[:, None], dimension_numbers=dnums, slice_sizes=(1,),
      mode=lax.GatherScatterMode.PROMISE_IN_BOUNDS)


def _take(mx, bi, omx, obi):
  take = (omx > mx) | ((omx == mx) & (obi < bi))
  return lax.select(take, omx, mx), lax.select(take, obi, bi)


def _lane_argmax(mx, bi, iota):
  for sh in (8, 4, 2, 1):
    idx = iota ^ sh
    mx, bi = _take(mx, bi, _shuffle(mx, idx), _shuffle(bi, idx))
  return bi


def _sc_body(x_hbm, start_hbm, end_hbm,
             b0, b1, b2, b3, mx_buf, bi_buf, res_buf,
             sh_mx, sh_bi, s0, s1, s2, s3):
  cid = lax.axis_index("c")
  sid = lax.axis_index("s")
  rblock = sid // NQ
  q = sid % NQ
  row0 = rblock * SUB
  col0 = cid * HALF + q * QCOLS
  qcol0 = q * QCOLS

  iota = lax.iota(jnp.int32, LANES)
  bufs = (b0, b1, b2, b3)
  sems = (s0, s1, s2, s3)

  def issue(c, b):
    pltpu.async_copy(
        x_hbm.at[pl.ds(row0, SUB), pl.ds(col0 + c * CHUNK_C, CHUNK_C)],
        bufs[b], sems[b])

  def drain(b):
    pltpu.make_async_copy(
        x_hbm.at[pl.ds(row0, SUB), pl.ds(col0, CHUNK_C)],
        bufs[b], sems[b]).wait()

  for b in range(NBUF):
    issue(b, b)

  neg = jnp.full((LANES,), NEG_INF, jnp.float32)
  zero = jnp.zeros((LANES,), jnp.int32)

  def chunk_fold(buf, chunk_col, mxs, bis):
    def step(g, carry):
      mxs_c, bis_c = carry
      new_mx = list(mxs_c)
      new_bi = list(bis_c)
      col = g * LANES
      cur = iota + (chunk_col + col)
      for s in range(SUB):
        v = buf[s, pl.ds(col, LANES)]
        m = v > new_mx[s]
        new_mx[s] = lax.select(m, v, new_mx[s])
        new_bi[s] = lax.select(m, cur, new_bi[s])
      return tuple(new_mx), tuple(new_bi)

    return lax.fori_loop(0, CHUNK_C // LANES, step, (mxs, bis))

  n_rounds = NCHUNKS // NBUF - 1

  def round_body(r, carry):
    mxs, bis = carry
    for b in range(NBUF):
      c = r * NBUF + b
      drain(b)
      mxs, bis = chunk_fold(bufs[b], qcol0 + c * CHUNK_C, mxs, bis)
      issue(c + NBUF, b)
    return mxs, bis

  mxs, bis = lax.fori_loop(
      0, n_rounds, round_body,
      (tuple([neg] * SUB), tuple([zero] * SUB)))

  for b in range(NBUF):
    c = (NCHUNKS - NBUF) + b
    drain(b)
    mxs, bis = chunk_fold(bufs[b], qcol0 + c * CHUNK_C, mxs, bis)
  mxs, bis = list(mxs), list(bis)

  for s in range(SUB):
    mx_buf[pl.ds(s * LANES, LANES)] = mxs[s]
    bi_buf[pl.ds(s * LANES, LANES)] = bis[s]
  slot = SUB * LANES
  pltpu.sync_copy(mx_buf, sh_mx.at[pl.ds(sid * slot, slot)])
  pltpu.sync_copy(bi_buf, sh_bi.at[pl.ds(sid * slot, slot)])
  plsc.subcore_barrier()

  @pl.when(q == 0)
  def _():
    mx_l = list(mxs)
    bi_l = list(bis)
    for dq in range(1, NQ):
      pltpu.sync_copy(sh_mx.at[pl.ds((sid + dq) * slot, slot)], mx_buf)
      pltpu.sync_copy(sh_bi.at[pl.ds((sid + dq) * slot, slot)], bi_buf)
      for s in range(SUB):
        mx_l[s], bi_l[s] = _take(
            mx_l[s], bi_l[s],
            mx_buf[pl.ds(s * LANES, LANES)],
            bi_buf[pl.ds(s * LANES, LANES)])
    res_v = jnp.zeros((LANES,), jnp.int32)
    for s in range(SUB):
      idx_v = _lane_argmax(mx_l[s], bi_l[s], iota)
      res_v = lax.select(iota == s, idx_v, res_v)
    res_buf[...] = res_v

    @pl.when(cid == 0)
    def _():
      pltpu.sync_copy(res_buf.at[pl.ds(0, SUB)],
                      start_hbm.at[pl.ds(row0, SUB)])

    @pl.when(cid == 1)
    def _():
      pltpu.sync_copy(res_buf.at[pl.ds(0, SUB)],
                      end_hbm.at[pl.ds(row0, SUB)])


def _sc_argmax(x):
  mesh = plsc.VectorSubcoreMesh(core_axis_name="c", subcore_axis_name="s")
  run = functools.partial(
      pl.kernel,
      out_type=(jax.ShapeDtypeStruct((R_SC,), jnp.int32),
                jax.ShapeDtypeStruct((R_SC,), jnp.int32)),
      mesh=mesh,
      scratch_types=(
          [pltpu.VMEM((SUB, CHUNK_C), jnp.float32)] * NBUF
          + [pltpu.VMEM((SUB * LANES,), jnp.float32),
             pltpu.VMEM((SUB * LANES,), jnp.int32),
             pltpu.VMEM((LANES,), jnp.int32),
             pltpu.VMEM_SHARED((16 * SUB * LANES,), jnp.float32),
             pltpu.VMEM_SHARED((16 * SUB * LANES,), jnp.int32)]
          + [pltpu.SemaphoreType.DMA] * NBUF
      ),
      compiler_params=pltpu.CompilerParams(use_tc_tiling_on_sc=True),
  )(_sc_body)
  return run(x)


def _tc_kernel(x_ref, out_ref):
  lane = lax.broadcasted_iota(jnp.int32, (TC_RB, 128), 1)
  neg = jnp.full((TC_RB, 128), NEG_INF, jnp.float32)
  zero = jnp.zeros((TC_RB, 128), jnp.int32)

  def step(i, carry):
    mxs, bis = carry
    new_mx = list(mxs)
    new_bi = list(bis)
    for u in range(TC_UNROLL):
      a = u % TC_NACC
      v = i * TC_UNROLL + u
      val = x_ref[:, pl.ds(v * 128, 128)]
      cur = lane + v * 128
      m = val > new_mx[a]
      new_mx[a] = jnp.where(m, val, new_mx[a])
      new_bi[a] = jnp.where(m, cur, new_bi[a])
    return tuple(new_mx), tuple(new_bi)

  n_iters = HALF // 128 // TC_UNROLL
  mxs, bis = lax.fori_loop(
      0, n_iters, step, (tuple([neg] * TC_NACC), tuple([zero] * TC_NACC)))

  mx, bi = mxs[0], bis[0]
  for a in range(1, TC_NACC):
    take = (mxs[a] > mx) | ((mxs[a] == mx) & (bis[a] < bi))
    mx = jnp.where(take, mxs[a], mx)
    bi = jnp.where(take, bis[a], bi)
  mbest = jnp.max(mx, axis=1, keepdims=True)
  cand = jnp.where(mx == mbest, bi, BIG_I32)
  idx = jnp.min(cand, axis=1).astype(jnp.int32)
  out_ref[...] = idx.reshape(1, 1, 1, TC_RB)


def _tc_argmax(x):
  return pl.pallas_call(
      _tc_kernel,
      grid=(2, TC_NR),
      in_specs=[pl.BlockSpec(
          (TC_RB, HALF),
          lambda h, r: (r + R_SC // TC_RB, h))],
      out_specs=pl.BlockSpec((1, 1, 1, TC_RB), lambda h, r: (h, r, 0, 0)),
      out_shape=jax.ShapeDtypeStruct((2, TC_NR, 1, TC_RB), jnp.int32),
  )(x)


@jax.jit
def _argmax_halves(x):
  sc_start, sc_end = _sc_argmax(x)
  tc_out = _tc_argmax(x)
  start = jnp.concatenate([sc_start, tc_out[0].reshape(R_TC)])
  end = jnp.concatenate([sc_end, tc_out[1].reshape(R_TC)])
  return start, end


def kernel(inputs):
  start, end = _argmax_halves(inputs)
  return (start, end)
